# baseline (speedup 1.0000x reference)
import jax
import jax.numpy as jnp
from jax.experimental import pallas as pl
from jax.experimental.pallas import tpu as pltpu

EPS_DICE = 0.0001
ROWS = 128


def _dice_body(out_ref, tgt_ref, loss_ref, acc_ref):
    i = pl.program_id(1)
    nblk = pl.num_programs(1)

    @pl.when(i == 0)
    def _init():
        acc_ref[...] = jnp.zeros_like(acc_ref)

    x = out_ref[0]
    t = tgt_ref[0]
    c = x.shape[0]
    sig = jax.nn.sigmoid(x)
    iota_c = jax.lax.broadcasted_iota(jnp.int32, x.shape, 0)
    mask = (iota_c == t[None, :, :]).astype(jnp.float32)
    acc_ref[0] += jnp.sum(sig, axis=1)
    acc_ref[1] += jnp.sum(sig * mask, axis=1)
    acc_ref[2] += jnp.sum(mask, axis=1)

    @pl.when(i == nblk - 1)
    def _fin():
        s = jnp.sum(acc_ref[0], axis=1)
        tt = jnp.sum(acc_ref[1], axis=1)
        n = jnp.sum(acc_ref[2], axis=1)
        per_class = 1.0 - 2.0 * tt / (s + n + EPS_DICE)
        loss_ref[0, 0, :] = jnp.full((loss_ref.shape[-1],), jnp.sum(per_class) / c)


def kernel(output, target):
    b, c, h, w = output.shape
    tgt = target.astype(jnp.int32)
    nblk = h // ROWS
    padded = pl.pallas_call(
        _dice_body,
        grid=(b, nblk),
        in_specs=[
            pl.BlockSpec((1, c, ROWS, w), lambda bi, i: (bi, 0, i, 0)),
            pl.BlockSpec((1, ROWS, w), lambda bi, i: (bi, i, 0)),
        ],
        out_specs=pl.BlockSpec((1, 1, 128), lambda bi, i: (bi, 0, 0)),
        out_shape=jax.ShapeDtypeStruct((b, 1, 128), jnp.float32),
        scratch_shapes=[pltpu.VMEM((3, c, w), jnp.float32)],
    )(output, tgt)
    return padded[:, 0, 0]
